# baseline (speedup 1.0000x reference)
import functools

import jax
import jax.numpy as jnp
from jax import lax
from jax.experimental import pallas as pl
from jax.experimental.pallas import tpu as pltpu
from jax.experimental.pallas import tpu_sc as plsc

_MASK_ID = 103
_NUM_CORES = 2
_NUM_SUBCORES = 16
_NUM_WORKERS = _NUM_CORES * _NUM_SUBCORES


def kernel(inputs_embeds, word_emb, prompt_emb):
    B, S, H = inputs_embeds.shape
    P = prompt_emb.shape[0]
    T = 1 + P + S

    assert _NUM_WORKERS % B == 0 and S % (_NUM_WORKERS // B) == 0
    wpb = _NUM_WORKERS // B
    rows = S // wpb

    mesh = plsc.VectorSubcoreMesh(core_axis_name="c", subcore_axis_name="s")

    in_flat = inputs_embeds.reshape(B * S * H)
    word_flat = word_emb.reshape(-1)
    prompt_flat = prompt_emb.reshape(P * H)

    @functools.partial(
        pl.kernel,
        out_type=jax.ShapeDtypeStruct((B * T * H,), inputs_embeds.dtype),
        mesh=mesh,
        scratch_types=[
            pltpu.SemaphoreType.DMA,
            pltpu.SemaphoreType.DMA,
            pltpu.SemaphoreType.DMA,
        ],
    )
    def body(in_hbm, word_hbm, prompt_hbm, out_hbm, sem_main, sem_p, sem_m):
        wid = lax.axis_index("s") * _NUM_CORES + lax.axis_index("c")
        b = wid // wpb
        r0 = (wid % wpb) * rows

        cp_main = pltpu.async_copy(
            in_hbm.at[pl.ds((b * S + r0) * H, rows * H)],
            out_hbm.at[pl.ds((b * T + 1 + P + r0) * H, rows * H)],
            sem_main,
        )

        @pl.when(wid < B)
        def _prefix():
            cp_p = pltpu.async_copy(
                prompt_hbm, out_hbm.at[pl.ds((wid * T + 1) * H, P * H)], sem_p
            )
            cp_m = pltpu.async_copy(
                word_hbm.at[pl.ds(_MASK_ID * H, H)],
                out_hbm.at[pl.ds(wid * T * H, H)],
                sem_m,
            )
            cp_p.wait()
            cp_m.wait()

        cp_main.wait()

    return body(in_flat, word_flat, prompt_flat).reshape(B, T, H)


# speedup vs baseline: 17.0570x; 17.0570x over previous
import jax
import jax.numpy as jnp
from jax.experimental import pallas as pl

_MASK_ID = 103
_HC = 8


def kernel(inputs_embeds, word_emb, prompt_emb):
    B, S, H = inputs_embeds.shape
    P = prompt_emb.shape[0]
    T = 1 + P + S
    Hc = H // _HC
    mb, mr = divmod(_MASK_ID, 8)

    def body(in_ref, word_ref, prompt_ref, out_ref):
        out_ref[0, pl.ds(0, 1), :] = word_ref[pl.ds(mr, 1), :]
        out_ref[0, pl.ds(1, P), :] = prompt_ref[...]
        out_ref[0, pl.ds(1 + P, S), :] = in_ref[0]

    return pl.pallas_call(
        body,
        grid=(B, _HC),
        in_specs=[
            pl.BlockSpec((1, S, Hc), lambda b, h: (b, 0, h)),
            pl.BlockSpec((8, Hc), lambda b, h: (mb, h)),
            pl.BlockSpec((P, Hc), lambda b, h: (0, h)),
        ],
        out_specs=pl.BlockSpec((1, T, Hc), lambda b, h: (b, 0, h)),
        out_shape=jax.ShapeDtypeStruct((B, T, H), inputs_embeds.dtype),
    )(inputs_embeds, word_emb, prompt_emb)
